# baseline (speedup 1.0000x reference)
import functools

import jax
import jax.numpy as jnp
from jax import lax
from jax.experimental import pallas as pl
from jax.experimental.pallas import tpu as pltpu
from jax.experimental.pallas import tpu_sc as plsc

NS = 16
LANES = 16
F = 128
NCHUNK = F // LANES
L_BASKET = 200
ROWS_PER_TILE = 16
N_FULL_TILES = L_BASKET // ROWS_PER_TILE
TAIL_ROWS = L_BASKET - N_FULL_TILES * ROWS_PER_TILE
TILE_TAIL = N_FULL_TILES
TILE_UI = TILE_TAIL + 1
N_PARTIALS = TILE_UI + 1
INV_L = 1.0 / L_BASKET

_mesh = plsc.VectorSubcoreMesh(
    core_axis_name="c", subcore_axis_name="s", num_cores=1, num_subcores=NS
)


@functools.partial(
    pl.kernel,
    out_type=jax.ShapeDtypeStruct((N_PARTIALS, LANES), jnp.float32),
    mesh=_mesh,
    scratch_types=[
        pltpu.VMEM((ROWS_PER_TILE,), jnp.int32),
        pltpu.VMEM((1,), jnp.int32),
        pltpu.VMEM((1,), jnp.int32),
        pltpu.VMEM((1,), jnp.int32),
        pltpu.VMEM((ROWS_PER_TILE, F), jnp.float32),
        pltpu.VMEM((1, F), jnp.float32),
        pltpu.VMEM((1, F), jnp.float32),
        pltpu.VMEM((1, F), jnp.float32),
        pltpu.VMEM((LANES,), jnp.float32),
        pltpu.SemaphoreType.DMA,
        pltpu.SemaphoreType.DMA,
        pltpu.SemaphoreType.DMA,
    ],
)
def _fpmc_sc(idx_hbm, i_hbm, j_hbm, u_hbm, vui_hbm, viu_hbm, vil_hbm, vli_hbm,
             out_hbm,
             idx_v, i_v, j_v, u_v, rows_v, row_a, row_b, row_c,
             part_v, sem0, sem1, sem2):
    s = lax.axis_index("s")

    def basket(nrows, base):
        ca = pltpu.async_copy(idx_hbm.at[pl.ds(base, nrows)],
                              idx_v.at[pl.ds(0, nrows)], sem0)
        ci = pltpu.async_copy(i_hbm, i_v, sem1)
        cj = pltpu.async_copy(j_hbm, j_v, sem2)
        ca.wait()
        ci.wait()
        cj.wait()
        cr = pltpu.async_copy(vli_hbm.at[idx_v.at[pl.ds(0, nrows)]],
                              rows_v.at[pl.ds(0, nrows)], sem0)
        cb = pltpu.async_copy(vil_hbm.at[i_v], row_a, sem1)
        cc = pltpu.async_copy(vil_hbm.at[j_v], row_b, sem2)
        cr.wait()
        cb.wait()
        cc.wait()

        def row_add(r, accs):
            return tuple(
                accs[k] + rows_v[r, pl.ds(k * LANES, LANES)]
                for k in range(NCHUNK)
            )

        accs = lax.fori_loop(
            1, nrows, row_add,
            tuple(rows_v[0, pl.ds(k * LANES, LANES)] for k in range(NCHUNK)),
        )
        part = jnp.zeros((LANES,), jnp.float32)
        for k in range(NCHUNK):
            dsl = pl.ds(k * LANES, LANES)
            part = part + (row_a[0, dsl] - row_b[0, dsl]) * accs[k]
        part_v[...] = part * INV_L
        pltpu.sync_copy(part_v, out_hbm.at[s])

    @pl.when(s < N_FULL_TILES)
    def _basket_full():
        basket(ROWS_PER_TILE, s * ROWS_PER_TILE)

    @pl.when(s == TILE_TAIL)
    def _basket_tail():
        basket(TAIL_ROWS, N_FULL_TILES * ROWS_PER_TILE)

    @pl.when(s == TILE_UI)
    def _user_item():
        cu = pltpu.async_copy(u_hbm, u_v, sem0)
        ci = pltpu.async_copy(i_hbm, i_v, sem1)
        cj = pltpu.async_copy(j_hbm, j_v, sem2)
        cu.wait()
        ci.wait()
        cj.wait()
        ca = pltpu.async_copy(vui_hbm.at[u_v], row_a, sem0)
        cb = pltpu.async_copy(viu_hbm.at[i_v], row_b, sem1)
        cc = pltpu.async_copy(viu_hbm.at[j_v], row_c, sem2)
        ca.wait()
        cb.wait()
        cc.wait()
        sv = jnp.zeros((LANES,), jnp.float32)
        for k in range(NCHUNK):
            dsl = pl.ds(k * LANES, LANES)
            sv = sv + row_a[0, dsl] * (row_b[0, dsl] - row_c[0, dsl])
        part_v[...] = sv
        pltpu.sync_copy(part_v, out_hbm.at[TILE_UI])


def kernel(u, i, j, b_tm1, VUI, VIU, VIL, VLI):
    idx = b_tm1.astype(jnp.int32)
    i1 = jnp.asarray(i, jnp.int32).reshape(1)
    j1 = jnp.asarray(j, jnp.int32).reshape(1)
    u1 = jnp.asarray(u, jnp.int32).reshape(1)
    parts = _fpmc_sc(idx, i1, j1, u1, VUI, VIU, VIL, VLI)
    d = jnp.sum(parts)
    return 1.0 / (1.0 + jnp.exp(d))


# speedup vs baseline: 1.1347x; 1.1347x over previous
import functools

import jax
import jax.numpy as jnp
from jax import lax
from jax.experimental import pallas as pl
from jax.experimental.pallas import tpu as pltpu
from jax.experimental.pallas import tpu_sc as plsc

NS = 16
LANES = 16
F = 128
NCHUNK = F // LANES
L_BASKET = 200
ROWS_PER_TILE = 16
N_FULL_TILES = L_BASKET // ROWS_PER_TILE
TAIL_ROWS = L_BASKET - N_FULL_TILES * ROWS_PER_TILE
TILE_TAIL = N_FULL_TILES
TILE_UI = TILE_TAIL + 1
TILE_IL = TILE_UI + 1
N_PARTIALS = TILE_IL + 1
INV_L = 1.0 / L_BASKET

_mesh = plsc.VectorSubcoreMesh(
    core_axis_name="c", subcore_axis_name="s", num_cores=1, num_subcores=NS
)


@functools.partial(
    pl.kernel,
    out_type=jax.ShapeDtypeStruct((LANES,), jnp.float32),
    mesh=_mesh,
    scratch_types=[
        pltpu.VMEM((ROWS_PER_TILE,), jnp.int32),
        pltpu.VMEM((1,), jnp.int32),
        pltpu.VMEM((1,), jnp.int32),
        pltpu.VMEM((1,), jnp.int32),
        pltpu.VMEM((ROWS_PER_TILE, F), jnp.float32),
        pltpu.VMEM_SHARED((N_PARTIALS, F), jnp.float32),
        pltpu.SemaphoreType.DMA,
        pltpu.SemaphoreType.DMA,
        pltpu.SemaphoreType.DMA,
    ],
)
def _fpmc_sc(idx_hbm, i_hbm, j_hbm, u_hbm, vui_hbm, viu_hbm, vil_hbm, vli_hbm,
             out_hbm,
             idx_v, i_v, j_v, u_v, rows_v, shared, sem0, sem1, sem2):
    s = lax.axis_index("s")

    def basket(nrows, base):
        pltpu.sync_copy(idx_hbm.at[pl.ds(base, nrows)],
                        idx_v.at[pl.ds(0, nrows)])
        pltpu.async_copy(vli_hbm.at[idx_v.at[pl.ds(0, nrows)]],
                         rows_v.at[pl.ds(0, nrows)], sem0).wait()

        def row_add(r, accs):
            return tuple(
                accs[k] + rows_v[r, pl.ds(k * LANES, LANES)]
                for k in range(NCHUNK)
            )

        accs = lax.fori_loop(
            1, nrows, row_add,
            tuple(rows_v[0, pl.ds(k * LANES, LANES)] for k in range(NCHUNK)),
        )
        for k in range(NCHUNK):
            rows_v[0, pl.ds(k * LANES, LANES)] = accs[k]
        pltpu.sync_copy(rows_v.at[pl.ds(0, 1)], shared.at[pl.ds(s, 1)])

    @pl.when(s < N_FULL_TILES)
    def _basket_full():
        basket(ROWS_PER_TILE, s * ROWS_PER_TILE)

    @pl.when(s == TILE_TAIL)
    def _basket_tail():
        basket(TAIL_ROWS, N_FULL_TILES * ROWS_PER_TILE)

    @pl.when(s == TILE_UI)
    def _user_item():
        cu = pltpu.async_copy(u_hbm, u_v, sem0)
        ci = pltpu.async_copy(i_hbm, i_v, sem1)
        cj = pltpu.async_copy(j_hbm, j_v, sem2)
        cu.wait()
        ci.wait()
        cj.wait()
        ca = pltpu.async_copy(vui_hbm.at[u_v], rows_v.at[pl.ds(0, 1)], sem0)
        cb = pltpu.async_copy(viu_hbm.at[i_v], rows_v.at[pl.ds(1, 1)], sem1)
        cc = pltpu.async_copy(viu_hbm.at[j_v], rows_v.at[pl.ds(2, 1)], sem2)
        ca.wait()
        cb.wait()
        cc.wait()
        sv = jnp.zeros((LANES,), jnp.float32)
        for k in range(NCHUNK):
            dsl = pl.ds(k * LANES, LANES)
            sv = sv + rows_v[0, dsl] * (rows_v[1, dsl] - rows_v[2, dsl])
        rows_v[3, pl.ds(0, LANES)] = sv
        pltpu.sync_copy(rows_v.at[3, pl.ds(0, LANES)],
                        shared.at[TILE_UI, pl.ds(0, LANES)])

    @pl.when(s == TILE_IL)
    def _item_diff():
        ci = pltpu.async_copy(i_hbm, i_v, sem1)
        cj = pltpu.async_copy(j_hbm, j_v, sem2)
        ci.wait()
        cj.wait()
        ca = pltpu.async_copy(vil_hbm.at[i_v], rows_v.at[pl.ds(0, 1)], sem1)
        cb = pltpu.async_copy(vil_hbm.at[j_v], rows_v.at[pl.ds(1, 1)], sem2)
        ca.wait()
        cb.wait()
        for k in range(NCHUNK):
            dsl = pl.ds(k * LANES, LANES)
            rows_v[2, dsl] = rows_v[0, dsl] - rows_v[1, dsl]
        pltpu.sync_copy(rows_v.at[pl.ds(2, 1)], shared.at[pl.ds(TILE_IL, 1)])

    plsc.subcore_barrier()

    @pl.when(s == 0)
    def _combine():
        pltpu.sync_copy(shared, rows_v.at[pl.ds(0, N_PARTIALS)])

        def row_add(t, ms):
            return tuple(
                ms[k] + rows_v[t, pl.ds(k * LANES, LANES)]
                for k in range(NCHUNK)
            )

        ms = lax.fori_loop(
            1, N_FULL_TILES + 1, row_add,
            tuple(rows_v[0, pl.ds(k * LANES, LANES)] for k in range(NCHUNK)),
        )
        tot = rows_v[TILE_UI, pl.ds(0, LANES)]
        for k in range(NCHUNK):
            tot = tot + rows_v[TILE_IL, pl.ds(k * LANES, LANES)] * (ms[k] * INV_L)
        lane = lax.iota(jnp.int32, LANES)
        for dist in (8, 4, 2, 1):
            perm = jnp.bitwise_xor(lane, dist)
            tot = tot + tot.at[perm].get(mode="promise_in_bounds")
        rows_v[15, pl.ds(0, LANES)] = 1.0 / (1.0 + jnp.exp(tot))
        pltpu.sync_copy(rows_v.at[15, pl.ds(0, LANES)], out_hbm)


def kernel(u, i, j, b_tm1, VUI, VIU, VIL, VLI):
    idx = b_tm1.astype(jnp.int32)
    i1 = jnp.asarray(i, jnp.int32).reshape(1)
    j1 = jnp.asarray(j, jnp.int32).reshape(1)
    u1 = jnp.asarray(u, jnp.int32).reshape(1)
    out = _fpmc_sc(idx, i1, j1, u1, VUI, VIU, VIL, VLI)
    return out[0]


# speedup vs baseline: 1.1391x; 1.0039x over previous
import functools

import jax
import jax.numpy as jnp
from jax import lax
from jax.experimental import pallas as pl
from jax.experimental.pallas import tpu as pltpu
from jax.experimental.pallas import tpu_sc as plsc

NS = 16
LANES = 16
F = 128
NCHUNK = F // LANES
L_BASKET = 200
ROWS_PER_TILE = 16
N_FULL_TILES = L_BASKET // ROWS_PER_TILE
TAIL_ROWS = L_BASKET - N_FULL_TILES * ROWS_PER_TILE
TILE_TAIL = N_FULL_TILES
TILE_UI = TILE_TAIL + 1
TILE_IL = TILE_UI + 1
N_PARTIALS = TILE_IL + 1
INV_L = 1.0 / L_BASKET

_mesh = plsc.VectorSubcoreMesh(
    core_axis_name="c", subcore_axis_name="s", num_cores=1, num_subcores=NS
)


@functools.partial(
    pl.kernel,
    out_type=jax.ShapeDtypeStruct((LANES,), jnp.float32),
    mesh=_mesh,
    scratch_types=[
        pltpu.VMEM((ROWS_PER_TILE,), jnp.int32),
        pltpu.VMEM((1,), jnp.int32),
        pltpu.VMEM((1,), jnp.int32),
        pltpu.VMEM((1,), jnp.int32),
        pltpu.VMEM((ROWS_PER_TILE, F), jnp.float32),
        pltpu.VMEM_SHARED((N_PARTIALS, F), jnp.float32),
        pltpu.SemaphoreType.DMA,
        pltpu.SemaphoreType.DMA,
        pltpu.SemaphoreType.DMA,
    ],
)
def _fpmc_sc(idx_hbm, i_hbm, j_hbm, u_hbm, vui_hbm, viu_hbm, vil_hbm, vli_hbm,
             out_hbm,
             idx_v, i_v, j_v, u_v, rows_v, shared, sem0, sem1, sem2):
    s = lax.axis_index("s")

    @pl.when(s <= TILE_TAIL)
    def _basket():
        base = jnp.minimum(s * ROWS_PER_TILE, L_BASKET - ROWS_PER_TILE)
        start = jnp.where(s == TILE_TAIL, ROWS_PER_TILE - TAIL_ROWS, 0)
        pltpu.sync_copy(idx_hbm.at[pl.ds(base, ROWS_PER_TILE)], idx_v)
        pltpu.async_copy(vli_hbm.at[idx_v], rows_v, sem0).wait()

        def row_add(r, accs):
            return tuple(
                accs[k] + rows_v[r, pl.ds(k * LANES, LANES)]
                for k in range(NCHUNK)
            )

        accs = lax.fori_loop(
            start, ROWS_PER_TILE, row_add,
            tuple(jnp.zeros((LANES,), jnp.float32) for _ in range(NCHUNK)),
        )
        for k in range(NCHUNK):
            rows_v[0, pl.ds(k * LANES, LANES)] = accs[k]
        pltpu.sync_copy(rows_v.at[pl.ds(0, 1)], shared.at[pl.ds(s, 1)])

    @pl.when(s == TILE_UI)
    def _singles():
        cu = pltpu.async_copy(u_hbm, u_v, sem0)
        ci = pltpu.async_copy(i_hbm, i_v, sem1)
        cj = pltpu.async_copy(j_hbm, j_v, sem2)
        cu.wait()
        ci.wait()
        cj.wait()
        c0 = pltpu.async_copy(vui_hbm.at[u_v], rows_v.at[pl.ds(0, 1)], sem0)
        c1 = pltpu.async_copy(viu_hbm.at[i_v], rows_v.at[pl.ds(1, 1)], sem1)
        c2 = pltpu.async_copy(viu_hbm.at[j_v], rows_v.at[pl.ds(2, 1)], sem2)
        c3 = pltpu.async_copy(vil_hbm.at[i_v], rows_v.at[pl.ds(4, 1)], sem1)
        c4 = pltpu.async_copy(vil_hbm.at[j_v], rows_v.at[pl.ds(5, 1)], sem2)
        c0.wait()
        c1.wait()
        c2.wait()
        c3.wait()
        c4.wait()
        sv = jnp.zeros((LANES,), jnp.float32)
        for k in range(NCHUNK):
            dsl = pl.ds(k * LANES, LANES)
            sv = sv + rows_v[0, dsl] * (rows_v[1, dsl] - rows_v[2, dsl])
            rows_v[6, dsl] = rows_v[4, dsl] - rows_v[5, dsl]
        rows_v[3, pl.ds(0, LANES)] = sv
        pltpu.sync_copy(rows_v.at[3, pl.ds(0, LANES)],
                        shared.at[TILE_UI, pl.ds(0, LANES)])
        pltpu.sync_copy(rows_v.at[pl.ds(6, 1)], shared.at[pl.ds(TILE_IL, 1)])

    plsc.subcore_barrier()

    @pl.when(s == 0)
    def _combine():
        pltpu.sync_copy(shared, rows_v.at[pl.ds(0, N_PARTIALS)])

        def row_add(t, ms):
            return tuple(
                ms[k] + rows_v[t, pl.ds(k * LANES, LANES)]
                for k in range(NCHUNK)
            )

        ms = lax.fori_loop(
            1, N_FULL_TILES + 1, row_add,
            tuple(rows_v[0, pl.ds(k * LANES, LANES)] for k in range(NCHUNK)),
        )
        tot = rows_v[TILE_UI, pl.ds(0, LANES)]
        for k in range(NCHUNK):
            tot = tot + rows_v[TILE_IL, pl.ds(k * LANES, LANES)] * (ms[k] * INV_L)
        lane = lax.iota(jnp.int32, LANES)
        for dist in (8, 4, 2, 1):
            perm = jnp.bitwise_xor(lane, dist)
            tot = tot + tot.at[perm].get(mode="promise_in_bounds")
        rows_v[15, pl.ds(0, LANES)] = 1.0 / (1.0 + jnp.exp(tot))
        pltpu.sync_copy(rows_v.at[15, pl.ds(0, LANES)], out_hbm)


def kernel(u, i, j, b_tm1, VUI, VIU, VIL, VLI):
    idx = b_tm1.astype(jnp.int32)
    i1 = jnp.asarray(i, jnp.int32).reshape(1)
    j1 = jnp.asarray(j, jnp.int32).reshape(1)
    u1 = jnp.asarray(u, jnp.int32).reshape(1)
    out = _fpmc_sc(idx, i1, j1, u1, VUI, VIU, VIL, VLI)
    return out[0]
